# Initial kernel scaffold; baseline (speedup 1.0000x reference)
import functools

import jax
import jax.numpy as jnp
from jax import lax
from jax.experimental import pallas as pl
from jax.experimental.pallas import tpu as pltpu
from jax.experimental.pallas import tpu_sc as plsc

N_NODES = 10000
D = 128
N_EDGES = 320000

_INFO = plsc.get_sparse_core_info()
NC = _INFO.num_cores
NS = _INFO.num_subcores
NW = NC * NS

CHUNK = 128
E_PER_W = N_EDGES // NW
FULL_CHUNKS = E_PER_W // CHUNK
TAIL = E_PER_W - FULL_CHUNKS * CHUNK
ROWS_PER_TILE = N_NODES // NS


def _sc_aggregate(x, src, dst, zeros):
    mesh = plsc.VectorSubcoreMesh(core_axis_name="c", subcore_axis_name="s")

    @functools.partial(
        pl.kernel,
        out_type=jax.ShapeDtypeStruct((NC, N_NODES, D), jnp.float32),
        mesh=mesh,
        scratch_types=[
            pltpu.VMEM_SHARED((N_NODES, D), jnp.float32),
            pltpu.VMEM((CHUNK,), jnp.int32),
            pltpu.VMEM((CHUNK,), jnp.int32),
            pltpu.VMEM((CHUNK, D), jnp.float32),
            pltpu.VMEM((TAIL,), jnp.int32),
            pltpu.VMEM((TAIL,), jnp.int32),
            pltpu.VMEM((TAIL, D), jnp.float32),
            pltpu.SemaphoreType.DMA,
        ],
    )
    def k(x_hbm, src_hbm, dst_hbm, z_hbm, out_hbm,
          acc, src_v, dst_v, rows_v, src_t, dst_t, rows_t, sem):
        c = lax.axis_index("c")
        s = lax.axis_index("s")
        wid = c * NS + s

        pltpu.sync_copy(z_hbm.at[pl.ds(s * ROWS_PER_TILE, ROWS_PER_TILE)],
                        acc.at[pl.ds(s * ROWS_PER_TILE, ROWS_PER_TILE)])
        plsc.subcore_barrier()

        base0 = wid * E_PER_W

        def body(i, carry):
            base = base0 + i * CHUNK
            pltpu.sync_copy(src_hbm.at[pl.ds(base, CHUNK)], src_v)
            pltpu.sync_copy(dst_hbm.at[pl.ds(base, CHUNK)], dst_v)
            pltpu.async_copy(x_hbm.at[src_v], rows_v, sem).wait()
            pltpu.sync_copy(rows_v, acc.at[dst_v], add=True)
            return carry

        lax.fori_loop(0, FULL_CHUNKS, body, 0)

        base = base0 + FULL_CHUNKS * CHUNK
        pltpu.sync_copy(src_hbm.at[pl.ds(base, TAIL)], src_t)
        pltpu.sync_copy(dst_hbm.at[pl.ds(base, TAIL)], dst_t)
        pltpu.async_copy(x_hbm.at[src_t], rows_t, sem).wait()
        pltpu.sync_copy(rows_t, acc.at[dst_t], add=True)

        plsc.subcore_barrier()
        pltpu.sync_copy(acc.at[pl.ds(s * ROWS_PER_TILE, ROWS_PER_TILE)],
                        out_hbm.at[c, pl.ds(s * ROWS_PER_TILE, ROWS_PER_TILE)])

    return k(x, src, dst, zeros)


_BLK = 1000


def _tc_body(p_ref, w_ref, b_ref, o_ref):
    agg = p_ref[0] + p_ref[1]
    o_ref[...] = (
        jnp.dot(agg, w_ref[...], preferred_element_type=jnp.float32) + b_ref[...]
    )


def _tc_project(partials, W, b2d):
    return pl.pallas_call(
        _tc_body,
        out_shape=jax.ShapeDtypeStruct((N_NODES, D), jnp.float32),
        grid=(N_NODES // _BLK,),
        in_specs=[
            pl.BlockSpec((NC, _BLK, D), lambda i: (0, i, 0)),
            pl.BlockSpec((D, D), lambda i: (0, 0)),
            pl.BlockSpec((1, D), lambda i: (0, 0)),
        ],
        out_specs=pl.BlockSpec((_BLK, D), lambda i: (i, 0)),
    )(partials, W, b2d)


def kernel(x, edge_index, W, b):
    ei = edge_index.astype(jnp.int32)
    dst = ei[0]
    src = ei[1]
    zeros = jnp.zeros((N_NODES, D), jnp.float32)
    partials = _sc_aggregate(x, src, dst, zeros)
    return _tc_project(partials, W, b.reshape(1, D))


# speedup vs baseline: 6.6802x; 6.6802x over previous
import functools

import jax
import jax.numpy as jnp
from jax import lax
from jax.experimental import pallas as pl
from jax.experimental.pallas import tpu as pltpu
from jax.experimental.pallas import tpu_sc as plsc

N_NODES = 10000
D = 128
N_EDGES = 320000

_INFO = plsc.get_sparse_core_info()
NC = _INFO.num_cores
NS = _INFO.num_subcores
NW = NC * NS

CHUNK = 128
E_PER_W = N_EDGES // NW
FULL_CHUNKS = E_PER_W // CHUNK
TAIL = E_PER_W - FULL_CHUNKS * CHUNK
N_PAD = 10240
ROWS_PER_TILE = N_PAD // NS


def _sc_aggregate(x, src, dst, zeros):
    mesh = plsc.VectorSubcoreMesh(core_axis_name="c", subcore_axis_name="s")

    @functools.partial(
        pl.kernel,
        out_type=jax.ShapeDtypeStruct((NC, N_PAD, D), jnp.float32),
        mesh=mesh,
        scratch_types=[
            pltpu.VMEM_SHARED((N_PAD, D), jnp.float32),
            pltpu.VMEM((CHUNK,), jnp.int32),
            pltpu.VMEM((CHUNK,), jnp.int32),
            pltpu.VMEM((CHUNK, D), jnp.float32),
            pltpu.VMEM((TAIL,), jnp.int32),
            pltpu.VMEM((TAIL,), jnp.int32),
            pltpu.VMEM((TAIL, D), jnp.float32),
            pltpu.SemaphoreType.DMA,
        ],
    )
    def k(x_hbm, src_hbm, dst_hbm, z_hbm, out_hbm,
          acc, src_v, dst_v, rows_v, src_t, dst_t, rows_t, sem):
        c = lax.axis_index("c")
        s = lax.axis_index("s")
        wid = c * NS + s

        pltpu.sync_copy(z_hbm.at[pl.ds(s * ROWS_PER_TILE, ROWS_PER_TILE)],
                        acc.at[pl.ds(s * ROWS_PER_TILE, ROWS_PER_TILE)])
        plsc.subcore_barrier()

        base0 = wid * E_PER_W

        def body(i, carry):
            base = base0 + i * CHUNK
            pltpu.sync_copy(src_hbm.at[pl.ds(base, CHUNK)], src_v)
            pltpu.sync_copy(dst_hbm.at[pl.ds(base, CHUNK)], dst_v)
            pltpu.async_copy(x_hbm.at[src_v], rows_v, sem).wait()
            pltpu.sync_copy(rows_v, acc.at[dst_v], add=True)
            return carry

        lax.fori_loop(0, FULL_CHUNKS, body, 0)

        base = base0 + FULL_CHUNKS * CHUNK
        pltpu.sync_copy(src_hbm.at[pl.ds(base, TAIL)], src_t)
        pltpu.sync_copy(dst_hbm.at[pl.ds(base, TAIL)], dst_t)
        pltpu.async_copy(x_hbm.at[src_t], rows_t, sem).wait()
        pltpu.sync_copy(rows_t, acc.at[dst_t], add=True)

        plsc.subcore_barrier()
        pltpu.sync_copy(acc.at[pl.ds(s * ROWS_PER_TILE, ROWS_PER_TILE)],
                        out_hbm.at[c, pl.ds(s * ROWS_PER_TILE, ROWS_PER_TILE)])

    return k(x, src, dst, zeros)


_BLK = 1000


def _tc_body(p_ref, w_ref, b_ref, o_ref):
    agg = p_ref[0] + p_ref[1]
    o_ref[...] = (
        jnp.dot(agg, w_ref[...], preferred_element_type=jnp.float32) + b_ref[...]
    )


def _tc_project(partials, W, b2d):
    return pl.pallas_call(
        _tc_body,
        out_shape=jax.ShapeDtypeStruct((N_NODES, D), jnp.float32),
        grid=(N_NODES // _BLK,),
        in_specs=[
            pl.BlockSpec((NC, _BLK, D), lambda i: (0, i, 0)),
            pl.BlockSpec((D, D), lambda i: (0, 0)),
            pl.BlockSpec((1, D), lambda i: (0, 0)),
        ],
        out_specs=pl.BlockSpec((_BLK, D), lambda i: (i, 0)),
    )(partials, W, b2d)


def kernel(x, edge_index, W, b):
    ei = edge_index.astype(jnp.int32)
    dst = ei[0]
    src = ei[1]
    zeros = jnp.zeros((N_PAD, D), jnp.float32)
    partials = _sc_aggregate(x, src, dst, zeros)
    return _tc_project(partials, W, b.reshape(1, D))


# speedup vs baseline: 12.3744x; 1.8524x over previous
import functools

import jax
import jax.numpy as jnp
from jax import lax
from jax.experimental import pallas as pl
from jax.experimental.pallas import tpu as pltpu
from jax.experimental.pallas import tpu_sc as plsc

N_NODES = 10000
D = 128
N_EDGES = 320000

_INFO = plsc.get_sparse_core_info()
NC = _INFO.num_cores
NS = _INFO.num_subcores
NW = NC * NS

CHUNK = 128
NCH = 80
NROW = 2
NIDX = 4
OUTER = NCH // NIDX
E_PAD = NW * NCH * CHUNK
N_PAD = 10240
ROWS_PER_TILE = N_PAD // NS


def _sc_aggregate(x, edges2d, zeros):
    mesh = plsc.VectorSubcoreMesh(core_axis_name="c", subcore_axis_name="s")

    @functools.partial(
        pl.kernel,
        out_type=jax.ShapeDtypeStruct((NC, N_PAD, D), jnp.float32),
        mesh=mesh,
        scratch_types=[
            pltpu.VMEM_SHARED((N_PAD, D), jnp.float32),
            pltpu.VMEM((NIDX, 2, CHUNK), jnp.int32),
            pltpu.VMEM((NROW, CHUNK, D), jnp.float32),
        ]
        + [pltpu.SemaphoreType.DMA] * (2 * NROW + NIDX),
    )
    def k(x_hbm, e_hbm, z_hbm, out_hbm, acc, ibufs, bufs, *sems):
        gs = sems[:NROW]
        ss = sems[NROW:2 * NROW]
        isem = sems[2 * NROW:]
        c = lax.axis_index("c")
        s = lax.axis_index("s")
        wid = c * NS + s

        pltpu.sync_copy(z_hbm.at[pl.ds(s * ROWS_PER_TILE, ROWS_PER_TILE)],
                        acc.at[pl.ds(s * ROWS_PER_TILE, ROWS_PER_TILE)])
        plsc.subcore_barrier()

        rb = wid * NCH

        def idx_wait(q):
            pltpu.make_async_copy(e_hbm.at[rb], ibufs.at[q], isem[q]).wait()

        def gather_wait(b):
            pltpu.make_async_copy(
                x_hbm.at[ibufs.at[0, 0]], bufs.at[b], gs[b]).wait()

        def scatter_wait(b):
            pltpu.make_async_copy(
                bufs.at[b], acc.at[ibufs.at[0, 1]], ss[b]).wait()

        for q in range(NIDX):
            pltpu.async_copy(e_hbm.at[rb + q], ibufs.at[q], isem[q])
        for b in range(NROW):
            idx_wait(b)
            pltpu.async_copy(x_hbm.at[ibufs.at[b, 0]], bufs.at[b], gs[b])

        def outer(j, carry):
            base = j * NIDX
            for q in range(NIDX):
                i = base + q
                b = q % NROW
                gather_wait(b)
                pltpu.async_copy(bufs.at[b], acc.at[ibufs.at[q, 1]], ss[b],
                                 add=True)
                scatter_wait(b)

                @pl.when(j < OUTER - 1)
                def _():
                    pltpu.async_copy(e_hbm.at[rb + i + NIDX], ibufs.at[q],
                                     isem[q])

                def fire_next_gather():
                    qn = (q + NROW) % NIDX
                    idx_wait(qn)
                    pltpu.async_copy(
                        x_hbm.at[ibufs.at[qn, 0]], bufs.at[b], gs[b])

                if q < NIDX - NROW:
                    fire_next_gather()
                else:
                    pl.when(j < OUTER - 1)(fire_next_gather)
            return carry

        lax.fori_loop(0, OUTER, outer, 0)

        plsc.subcore_barrier()
        pltpu.sync_copy(acc.at[pl.ds(s * ROWS_PER_TILE, ROWS_PER_TILE)],
                        out_hbm.at[c, pl.ds(s * ROWS_PER_TILE, ROWS_PER_TILE)])

    return k(x, edges2d, zeros)


_BLK = 1000


def _tc_body(p_ref, w_ref, b_ref, o_ref):
    agg = p_ref[0] + p_ref[1]
    o_ref[...] = (
        jnp.dot(agg, w_ref[...], preferred_element_type=jnp.float32) + b_ref[...]
    )


def _tc_project(partials, W, b2d):
    return pl.pallas_call(
        _tc_body,
        out_shape=jax.ShapeDtypeStruct((N_NODES, D), jnp.float32),
        grid=(N_NODES // _BLK,),
        in_specs=[
            pl.BlockSpec((NC, _BLK, D), lambda i: (0, i, 0)),
            pl.BlockSpec((D, D), lambda i: (0, 0)),
            pl.BlockSpec((1, D), lambda i: (0, 0)),
        ],
        out_specs=pl.BlockSpec((_BLK, D), lambda i: (i, 0)),
    )(partials, W, b2d)


def kernel(x, edge_index, W, b):
    ei = edge_index.astype(jnp.int32)
    dst = ei[0]
    src = ei[1]
    n_extra = E_PAD - N_EDGES
    fill = jnp.arange(n_extra, dtype=jnp.int32)
    src = jnp.concatenate([src, fill % N_NODES])
    dst = jnp.concatenate([dst, N_NODES + fill % (N_PAD - N_NODES)])
    edges2d = jnp.stack(
        [src.reshape(E_PAD // CHUNK, CHUNK), dst.reshape(E_PAD // CHUNK, CHUNK)],
        axis=1)
    zeros = jnp.zeros((N_PAD, D), jnp.float32)
    partials = _sc_aggregate(x, edges2d, zeros)
    return _tc_project(partials, W, b.reshape(1, D))


# speedup vs baseline: 14.6068x; 1.1804x over previous
import functools

import jax
import jax.numpy as jnp
from jax import lax
from jax.experimental import pallas as pl
from jax.experimental.pallas import tpu as pltpu
from jax.experimental.pallas import tpu_sc as plsc

N_NODES = 10000
D = 128
N_EDGES = 320000

_INFO = plsc.get_sparse_core_info()
NC = _INFO.num_cores
NS = _INFO.num_subcores
NW = NC * NS

CHUNK = 128
N_CHUNKS = N_EDGES // CHUNK
NCH = N_CHUNKS // NW
N_EXTRA = N_CHUNKS - NCH * NW
NROW = 3
NIDX = 6
OUTER = NCH // NIDX
SLAB = 624
SLAB_REM = N_NODES - NS * SLAB


def _sc_aggregate(x, edges2d, zeros):
    mesh = plsc.VectorSubcoreMesh(core_axis_name="c", subcore_axis_name="s")

    @functools.partial(
        pl.kernel,
        out_type=jax.ShapeDtypeStruct((NC, N_NODES, D), jnp.float32),
        mesh=mesh,
        scratch_types=[
            pltpu.VMEM_SHARED((N_NODES, D), jnp.float32),
            pltpu.VMEM((NIDX, 2, CHUNK), jnp.int32),
            pltpu.VMEM((NROW, CHUNK, D), jnp.float32),
        ]
        + [pltpu.SemaphoreType.DMA] * (2 * NROW + NIDX),
    )
    def k(x_hbm, e_hbm, z_hbm, out_hbm, acc, ibufs, bufs, *sems):
        gs = sems[:NROW]
        ss = sems[NROW:2 * NROW]
        isem = sems[2 * NROW:]
        c = lax.axis_index("c")
        s = lax.axis_index("s")
        wid = c * NS + s

        pltpu.sync_copy(z_hbm.at[pl.ds(s * SLAB, SLAB)],
                        acc.at[pl.ds(s * SLAB, SLAB)])

        @pl.when(s == NS - 1)
        def _():
            pltpu.sync_copy(z_hbm.at[pl.ds(NS * SLAB, SLAB_REM)],
                            acc.at[pl.ds(NS * SLAB, SLAB_REM)])

        plsc.subcore_barrier()

        cb0 = wid * NCH

        def idx_wait(q):
            pltpu.make_async_copy(e_hbm.at[cb0], ibufs.at[q], isem[q]).wait()

        def gather_wait(b):
            pltpu.make_async_copy(
                x_hbm.at[ibufs.at[0, 0]], bufs.at[b], gs[b]).wait()

        def scatter_wait(b):
            pltpu.make_async_copy(
                bufs.at[b], acc.at[ibufs.at[0, 1]], ss[b]).wait()

        for q in range(NIDX):
            pltpu.async_copy(e_hbm.at[cb0 + q], ibufs.at[q], isem[q])
        for b in range(NROW):
            idx_wait(b)
            pltpu.async_copy(x_hbm.at[ibufs.at[b, 0]], bufs.at[b], gs[b])

        def outer(j, carry):
            base = j * NIDX
            for q in range(NIDX):
                i = base + q
                b = q % NROW
                gather_wait(b)
                pltpu.async_copy(bufs.at[b], acc.at[ibufs.at[q, 1]], ss[b],
                                 add=True)
                scatter_wait(b)

                @pl.when(j < OUTER - 1)
                def _():
                    pltpu.async_copy(e_hbm.at[cb0 + i + NIDX], ibufs.at[q],
                                     isem[q])

                def fire_next_gather():
                    qn = (q + NROW) % NIDX
                    idx_wait(qn)
                    pltpu.async_copy(
                        x_hbm.at[ibufs.at[qn, 0]], bufs.at[b], gs[b])

                if q < NIDX - NROW:
                    fire_next_gather()
                else:
                    pl.when(j < OUTER - 1)(fire_next_gather)
            return carry

        lax.fori_loop(0, OUTER, outer, 0)

        @pl.when(wid < N_EXTRA)
        def _():
            cb = NW * NCH + wid
            pltpu.sync_copy(e_hbm.at[cb], ibufs.at[0])
            pltpu.async_copy(x_hbm.at[ibufs.at[0, 0]], bufs.at[0], gs[0]).wait()
            pltpu.sync_copy(bufs.at[0], acc.at[ibufs.at[0, 1]], add=True)

        plsc.subcore_barrier()
        pltpu.sync_copy(acc.at[pl.ds(s * SLAB, SLAB)],
                        out_hbm.at[c, pl.ds(s * SLAB, SLAB)])

        @pl.when(s == NS - 1)
        def _():
            pltpu.sync_copy(acc.at[pl.ds(NS * SLAB, SLAB_REM)],
                            out_hbm.at[c, pl.ds(NS * SLAB, SLAB_REM)])

    return k(x, edges2d, zeros)


_BLK = 1000


def _tc_body(p_ref, w_ref, b_ref, o_ref):
    agg = p_ref[0] + p_ref[1]
    o_ref[...] = (
        jnp.dot(agg, w_ref[...], preferred_element_type=jnp.float32) + b_ref[...]
    )


def _tc_project(partials, W, b2d):
    return pl.pallas_call(
        _tc_body,
        out_shape=jax.ShapeDtypeStruct((N_NODES, D), jnp.float32),
        grid=(N_NODES // _BLK,),
        in_specs=[
            pl.BlockSpec((NC, _BLK, D), lambda i: (0, i, 0)),
            pl.BlockSpec((D, D), lambda i: (0, 0)),
            pl.BlockSpec((1, D), lambda i: (0, 0)),
        ],
        out_specs=pl.BlockSpec((_BLK, D), lambda i: (i, 0)),
    )(partials, W, b2d)


def kernel(x, edge_index, W, b):
    ei = edge_index.astype(jnp.int32)
    dst = ei[0]
    src = ei[1]
    edges2d = jnp.stack(
        [src.reshape(N_CHUNKS, CHUNK), dst.reshape(N_CHUNKS, CHUNK)], axis=1)
    zeros = jnp.zeros((N_NODES, D), jnp.float32)
    partials = _sc_aggregate(x, edges2d, zeros)
    return _tc_project(partials, W, b.reshape(1, D))


# speedup vs baseline: 15.9383x; 1.0912x over previous
import functools

import jax
import jax.numpy as jnp
from jax import lax
from jax.experimental import pallas as pl
from jax.experimental.pallas import tpu as pltpu
from jax.experimental.pallas import tpu_sc as plsc

N_NODES = 10000
D = 128
N_EDGES = 320000

_INFO = plsc.get_sparse_core_info()
NC = _INFO.num_cores
NS = _INFO.num_subcores
NW = NC * NS
NL = _INFO.num_lanes

CHUNK = 128
N_CHUNKS = N_EDGES // CHUNK
NCH = N_CHUNKS // NW
N_EXTRA = N_CHUNKS - NCH * NW
NROW = 3
NIDX = 6
OUTER = NCH // NIDX
SLAB = 624
SLAB_REM = N_NODES - NS * SLAB


def _sc_aggregate(x, edge_index):
    mesh = plsc.VectorSubcoreMesh(core_axis_name="c", subcore_axis_name="s")

    @functools.partial(
        pl.kernel,
        out_type=jax.ShapeDtypeStruct((NC, N_NODES, D), jnp.float32),
        mesh=mesh,
        scratch_types=[
            pltpu.VMEM_SHARED((N_NODES, D), jnp.float32),
            pltpu.VMEM((NIDX, 2, CHUNK), jnp.int32),
            pltpu.VMEM((NROW, CHUNK, D), jnp.float32),
        ]
        + [pltpu.SemaphoreType.DMA] * (2 * NROW + NIDX),
    )
    def k(x_hbm, e_hbm, out_hbm, acc, ibufs, bufs, *sems):
        gs = sems[:NROW]
        ss = sems[NROW:2 * NROW]
        isem = sems[2 * NROW:]
        c = lax.axis_index("c")
        s = lax.axis_index("s")
        wid = c * NS + s

        zv = jnp.zeros((NL,), jnp.float32)

        def zrow(r, carry):
            for l in range(D // NL):
                bufs[0, r, pl.ds(l * NL, NL)] = zv
            return carry

        lax.fori_loop(0, CHUNK, zrow, 0)
        for kk in range(SLAB // CHUNK):
            pltpu.sync_copy(bufs.at[0],
                            acc.at[pl.ds(s * SLAB + kk * CHUNK, CHUNK)])
        rem = SLAB - (SLAB // CHUNK) * CHUNK
        pltpu.sync_copy(bufs.at[0, pl.ds(0, rem)],
                        acc.at[pl.ds(s * SLAB + SLAB - rem, rem)])

        @pl.when(s == NS - 1)
        def _():
            pltpu.sync_copy(bufs.at[0, pl.ds(0, SLAB_REM)],
                            acc.at[pl.ds(NS * SLAB, SLAB_REM)])

        plsc.subcore_barrier()

        cb0 = wid * NCH

        def idx_load(chunk, q):
            pltpu.async_copy(e_hbm.at[0, pl.ds(chunk * CHUNK, CHUNK)],
                             ibufs.at[q, 1], isem[q])
            pltpu.async_copy(e_hbm.at[1, pl.ds(chunk * CHUNK, CHUNK)],
                             ibufs.at[q, 0], isem[q])

        def idx_wait(q):
            pltpu.make_async_copy(
                e_hbm.at[pl.ds(0, 2), pl.ds(0, CHUNK)], ibufs.at[q],
                isem[q]).wait()

        def gather_wait(b):
            pltpu.make_async_copy(
                x_hbm.at[ibufs.at[0, 0]], bufs.at[b], gs[b]).wait()

        def scatter_wait(b):
            pltpu.make_async_copy(
                bufs.at[b], acc.at[ibufs.at[0, 1]], ss[b]).wait()

        for q in range(NIDX):
            idx_load(cb0 + q, q)
        for b in range(NROW):
            idx_wait(b)
            pltpu.async_copy(x_hbm.at[ibufs.at[b, 0]], bufs.at[b], gs[b])

        def outer(j, carry):
            base = j * NIDX
            for q in range(NIDX):
                i = base + q
                b = q % NROW
                gather_wait(b)
                pltpu.async_copy(bufs.at[b], acc.at[ibufs.at[q, 1]], ss[b],
                                 add=True)
                scatter_wait(b)

                @pl.when(j < OUTER - 1)
                def _():
                    idx_load(cb0 + i + NIDX, q)

                def fire_next_gather():
                    qn = (q + NROW) % NIDX
                    idx_wait(qn)
                    pltpu.async_copy(
                        x_hbm.at[ibufs.at[qn, 0]], bufs.at[b], gs[b])

                if q < NIDX - NROW:
                    fire_next_gather()
                else:
                    pl.when(j < OUTER - 1)(fire_next_gather)
            return carry

        lax.fori_loop(0, OUTER, outer, 0)

        @pl.when(wid < N_EXTRA)
        def _():
            cb = NW * NCH + wid
            idx_load(cb, 0)
            idx_wait(0)
            pltpu.async_copy(x_hbm.at[ibufs.at[0, 0]], bufs.at[0], gs[0]).wait()
            pltpu.sync_copy(bufs.at[0], acc.at[ibufs.at[0, 1]], add=True)

        plsc.subcore_barrier()
        pltpu.sync_copy(acc.at[pl.ds(s * SLAB, SLAB)],
                        out_hbm.at[c, pl.ds(s * SLAB, SLAB)])

        @pl.when(s == NS - 1)
        def _():
            pltpu.sync_copy(acc.at[pl.ds(NS * SLAB, SLAB_REM)],
                            out_hbm.at[c, pl.ds(NS * SLAB, SLAB_REM)])

    return k(x, edge_index)


_BLK = 1000


def _tc_body(p_ref, w_ref, b_ref, o_ref):
    agg = p_ref[0] + p_ref[1]
    o_ref[...] = (
        jnp.dot(agg, w_ref[...], preferred_element_type=jnp.float32) + b_ref[...]
    )


def _tc_project(partials, W, b2d):
    return pl.pallas_call(
        _tc_body,
        out_shape=jax.ShapeDtypeStruct((N_NODES, D), jnp.float32),
        grid=(N_NODES // _BLK,),
        in_specs=[
            pl.BlockSpec((NC, _BLK, D), lambda i: (0, i, 0)),
            pl.BlockSpec((D, D), lambda i: (0, 0)),
            pl.BlockSpec((1, D), lambda i: (0, 0)),
        ],
        out_specs=pl.BlockSpec((_BLK, D), lambda i: (i, 0)),
    )(partials, W, b2d)


def kernel(x, edge_index, W, b):
    partials = _sc_aggregate(x, edge_index.astype(jnp.int32))
    return _tc_project(partials, W, b.reshape(1, D))


# speedup vs baseline: 16.4292x; 1.0308x over previous
import functools

import jax
import jax.numpy as jnp
from jax import lax
from jax.experimental import pallas as pl
from jax.experimental.pallas import tpu as pltpu
from jax.experimental.pallas import tpu_sc as plsc

N_NODES = 10000
D = 128
N_EDGES = 320000

_INFO = plsc.get_sparse_core_info()
NC = _INFO.num_cores
NS = _INFO.num_subcores
NW = NC * NS
NL = _INFO.num_lanes

CHUNK = 128
N_CHUNKS = N_EDGES // CHUNK
NCH = N_CHUNKS // NW
N_EXTRA = N_CHUNKS - NCH * NW
NROW = 3
NIDX = 6
OUTER = NCH // NIDX
SLAB = 624
SLAB_REM = N_NODES - NS * SLAB


def _sc_aggregate(x, edge_index):
    mesh = plsc.VectorSubcoreMesh(core_axis_name="c", subcore_axis_name="s")

    @functools.partial(
        pl.kernel,
        out_type=jax.ShapeDtypeStruct((NC, N_NODES, D), jnp.float32),
        mesh=mesh,
        scratch_types=[
            pltpu.VMEM_SHARED((N_NODES, D), jnp.float32),
            pltpu.VMEM((NIDX, 2, CHUNK), jnp.int32),
            pltpu.VMEM((NROW, CHUNK, D), jnp.float32),
        ]
        + [pltpu.SemaphoreType.DMA] * (2 * NROW + NIDX),
    )
    def k(x_hbm, e_hbm, out_hbm, acc, ibufs, bufs, *sems):
        gs = sems[:NROW]
        ss = sems[NROW:2 * NROW]
        isem = sems[2 * NROW:]
        c = lax.axis_index("c")
        s = lax.axis_index("s")
        wid = c * NS + s

        cb0 = wid * NCH

        def idx_load(chunk, q):
            pltpu.async_copy(e_hbm.at[0, pl.ds(chunk * CHUNK, CHUNK)],
                             ibufs.at[q, 1], isem[q])
            pltpu.async_copy(e_hbm.at[1, pl.ds(chunk * CHUNK, CHUNK)],
                             ibufs.at[q, 0], isem[q])

        def idx_wait(q):
            pltpu.make_async_copy(
                e_hbm.at[pl.ds(0, 2), pl.ds(0, CHUNK)], ibufs.at[q],
                isem[q]).wait()

        def gather_wait(b):
            pltpu.make_async_copy(
                x_hbm.at[ibufs.at[0, 0]], bufs.at[b], gs[b]).wait()

        def scatter_wait(b):
            pltpu.make_async_copy(
                bufs.at[b], acc.at[ibufs.at[0, 1]], ss[b]).wait()

        for q in range(NIDX):
            idx_load(cb0 + q, q)

        zv = jnp.zeros((NL,), jnp.float32)

        def zrow(r, carry):
            for l in range(D // NL):
                bufs[NROW - 1, r, pl.ds(l * NL, NL)] = zv
            return carry

        lax.fori_loop(0, CHUNK, zrow, 0)
        ZREM = SLAB - (SLAB // CHUNK) * CHUNK
        for kk in range(SLAB // CHUNK):
            pltpu.async_copy(bufs.at[NROW - 1],
                             acc.at[pl.ds(s * SLAB + kk * CHUNK, CHUNK)],
                             ss[0])
        pltpu.async_copy(bufs.at[NROW - 1, pl.ds(0, ZREM)],
                         acc.at[pl.ds(s * SLAB + SLAB - ZREM, ZREM)], ss[0])

        @pl.when(s == NS - 1)
        def _():
            pltpu.async_copy(bufs.at[NROW - 1, pl.ds(0, SLAB_REM)],
                             acc.at[pl.ds(NS * SLAB, SLAB_REM)], ss[0])

        for kk in range(SLAB // CHUNK):
            pltpu.make_async_copy(
                bufs.at[NROW - 1], acc.at[pl.ds(s * SLAB, CHUNK)],
                ss[0]).wait()
        pltpu.make_async_copy(
            bufs.at[NROW - 1, pl.ds(0, ZREM)], acc.at[pl.ds(0, ZREM)],
            ss[0]).wait()

        @pl.when(s == NS - 1)
        def _():
            pltpu.make_async_copy(
                bufs.at[NROW - 1, pl.ds(0, SLAB_REM)],
                acc.at[pl.ds(0, SLAB_REM)], ss[0]).wait()

        for b in range(NROW):
            idx_wait(b)
            pltpu.async_copy(x_hbm.at[ibufs.at[b, 0]], bufs.at[b], gs[b])

        plsc.subcore_barrier()

        def outer(j, carry):
            base = j * NIDX
            for q in range(NIDX):
                i = base + q
                b = q % NROW
                gather_wait(b)
                pltpu.async_copy(bufs.at[b], acc.at[ibufs.at[q, 1]], ss[b],
                                 add=True)
                scatter_wait(b)

                @pl.when(j < OUTER - 1)
                def _():
                    idx_load(cb0 + i + NIDX, q)

                def fire_next_gather():
                    qn = (q + NROW) % NIDX
                    idx_wait(qn)
                    pltpu.async_copy(
                        x_hbm.at[ibufs.at[qn, 0]], bufs.at[b], gs[b])

                if q < NIDX - NROW:
                    fire_next_gather()
                else:
                    pl.when(j < OUTER - 1)(fire_next_gather)
            return carry

        lax.fori_loop(0, OUTER, outer, 0)

        @pl.when(wid < N_EXTRA)
        def _():
            cb = NW * NCH + wid
            idx_load(cb, 0)
            idx_wait(0)
            pltpu.async_copy(x_hbm.at[ibufs.at[0, 0]], bufs.at[0], gs[0]).wait()
            pltpu.sync_copy(bufs.at[0], acc.at[ibufs.at[0, 1]], add=True)

        plsc.subcore_barrier()
        pltpu.sync_copy(acc.at[pl.ds(s * SLAB, SLAB)],
                        out_hbm.at[c, pl.ds(s * SLAB, SLAB)])

        @pl.when(s == NS - 1)
        def _():
            pltpu.sync_copy(acc.at[pl.ds(NS * SLAB, SLAB_REM)],
                            out_hbm.at[c, pl.ds(NS * SLAB, SLAB_REM)])

    return k(x, edge_index)


_BLK = 2000


def _tc_body(p_ref, w_ref, b_ref, o_ref):
    agg = p_ref[0] + p_ref[1]
    o_ref[...] = (
        jnp.dot(agg, w_ref[...], preferred_element_type=jnp.float32) + b_ref[...]
    )


def _tc_project(partials, W, b2d):
    return pl.pallas_call(
        _tc_body,
        out_shape=jax.ShapeDtypeStruct((N_NODES, D), jnp.float32),
        grid=(N_NODES // _BLK,),
        in_specs=[
            pl.BlockSpec((NC, _BLK, D), lambda i: (0, i, 0)),
            pl.BlockSpec((D, D), lambda i: (0, 0)),
            pl.BlockSpec((1, D), lambda i: (0, 0)),
        ],
        out_specs=pl.BlockSpec((_BLK, D), lambda i: (i, 0)),
    )(partials, W, b2d)


def kernel(x, edge_index, W, b):
    partials = _sc_aggregate(x, edge_index.astype(jnp.int32))
    return _tc_project(partials, W, b.reshape(1, D))


# speedup vs baseline: 16.4798x; 1.0031x over previous
import functools

import jax
import jax.numpy as jnp
from jax import lax
from jax.experimental import pallas as pl
from jax.experimental.pallas import tpu as pltpu
from jax.experimental.pallas import tpu_sc as plsc

N_NODES = 10000
D = 128
N_EDGES = 320000

_INFO = plsc.get_sparse_core_info()
NC = _INFO.num_cores
NS = _INFO.num_subcores
NW = NC * NS
NL = _INFO.num_lanes

CHUNK = 128
N_CHUNKS = N_EDGES // CHUNK
NCH = N_CHUNKS // NW
N_EXTRA = N_CHUNKS - NCH * NW
NROW = 3
NIDX = 6
OUTER = NCH // NIDX
SLAB = 624
SLAB_REM = N_NODES - NS * SLAB


def _sc_aggregate(x, edge_index):
    mesh = plsc.VectorSubcoreMesh(core_axis_name="c", subcore_axis_name="s")

    @functools.partial(
        pl.kernel,
        out_type=jax.ShapeDtypeStruct((NC, N_NODES, D), jnp.float32),
        mesh=mesh,
        scratch_types=[
            pltpu.VMEM_SHARED((N_NODES, D), jnp.float32),
            pltpu.VMEM((NIDX, 2, CHUNK), jnp.int32),
            pltpu.VMEM((NROW, CHUNK, D), jnp.float32),
        ]
        + [pltpu.SemaphoreType.DMA] * (2 * NROW + NIDX),
    )
    def k(x_hbm, e_hbm, out_hbm, acc, ibufs, bufs, *sems):
        gs = sems[:NROW]
        ss = sems[NROW:2 * NROW]
        isem = sems[2 * NROW:]
        c = lax.axis_index("c")
        s = lax.axis_index("s")
        wid = s * NC + c

        cb0 = wid * NCH

        def idx_load(chunk, q):
            pltpu.async_copy(e_hbm.at[0, pl.ds(chunk * CHUNK, CHUNK)],
                             ibufs.at[q, 1], isem[q])
            pltpu.async_copy(e_hbm.at[1, pl.ds(chunk * CHUNK, CHUNK)],
                             ibufs.at[q, 0], isem[q])

        def idx_wait(q):
            pltpu.make_async_copy(
                e_hbm.at[pl.ds(0, 2), pl.ds(0, CHUNK)], ibufs.at[q],
                isem[q]).wait()

        def gather_wait(b):
            pltpu.make_async_copy(
                x_hbm.at[ibufs.at[0, 0]], bufs.at[b], gs[b]).wait()

        def scatter_wait(b):
            pltpu.make_async_copy(
                bufs.at[b], acc.at[ibufs.at[0, 1]], ss[b]).wait()

        for q in range(NIDX):
            idx_load(cb0 + q, q)

        zv = jnp.zeros((NL,), jnp.float32)

        def zrow(r, carry):
            for l in range(D // NL):
                bufs[NROW - 1, r, pl.ds(l * NL, NL)] = zv
            return carry

        lax.fori_loop(0, CHUNK, zrow, 0)
        ZREM = SLAB - (SLAB // CHUNK) * CHUNK
        for kk in range(SLAB // CHUNK):
            pltpu.async_copy(bufs.at[NROW - 1],
                             acc.at[pl.ds(s * SLAB + kk * CHUNK, CHUNK)],
                             ss[0])
        pltpu.async_copy(bufs.at[NROW - 1, pl.ds(0, ZREM)],
                         acc.at[pl.ds(s * SLAB + SLAB - ZREM, ZREM)], ss[0])

        @pl.when(s == NS - 1)
        def _():
            pltpu.async_copy(bufs.at[NROW - 1, pl.ds(0, SLAB_REM)],
                             acc.at[pl.ds(NS * SLAB, SLAB_REM)], ss[0])

        for kk in range(SLAB // CHUNK):
            pltpu.make_async_copy(
                bufs.at[NROW - 1], acc.at[pl.ds(s * SLAB, CHUNK)],
                ss[0]).wait()
        pltpu.make_async_copy(
            bufs.at[NROW - 1, pl.ds(0, ZREM)], acc.at[pl.ds(0, ZREM)],
            ss[0]).wait()

        @pl.when(s == NS - 1)
        def _():
            pltpu.make_async_copy(
                bufs.at[NROW - 1, pl.ds(0, SLAB_REM)],
                acc.at[pl.ds(0, SLAB_REM)], ss[0]).wait()

        for b in range(NROW):
            idx_wait(b)
            pltpu.async_copy(x_hbm.at[ibufs.at[b, 0]], bufs.at[b], gs[b])

        plsc.subcore_barrier()

        def outer(j, carry):
            base = j * NIDX
            for q in range(NIDX):
                i = base + q
                b = q % NROW
                gather_wait(b)
                pltpu.async_copy(bufs.at[b], acc.at[ibufs.at[q, 1]], ss[b],
                                 add=True)
                scatter_wait(b)

                @pl.when(j < OUTER - 1)
                def _():
                    idx_load(cb0 + i + NIDX, q)

                def fire_next_gather():
                    qn = (q + NROW) % NIDX
                    idx_wait(qn)
                    pltpu.async_copy(
                        x_hbm.at[ibufs.at[qn, 0]], bufs.at[b], gs[b])

                if q < NIDX - NROW:
                    fire_next_gather()
                else:
                    pl.when(j < OUTER - 1)(fire_next_gather)
            return carry

        lax.fori_loop(0, OUTER, outer, 0)

        @pl.when(wid < N_EXTRA)
        def _():
            cb = NW * NCH + wid
            idx_load(cb, 0)
            idx_wait(0)
            pltpu.async_copy(x_hbm.at[ibufs.at[0, 0]], bufs.at[0], gs[0]).wait()
            pltpu.sync_copy(bufs.at[0], acc.at[ibufs.at[0, 1]], add=True)

        plsc.subcore_barrier()
        pltpu.sync_copy(acc.at[pl.ds(s * SLAB, SLAB)],
                        out_hbm.at[c, pl.ds(s * SLAB, SLAB)])

        @pl.when(s == NS - 1)
        def _():
            pltpu.sync_copy(acc.at[pl.ds(NS * SLAB, SLAB_REM)],
                            out_hbm.at[c, pl.ds(NS * SLAB, SLAB_REM)])

    return k(x, edge_index)


_BLK = 2000


def _tc_body(p_ref, w_ref, b_ref, o_ref):
    agg = p_ref[0] + p_ref[1]
    o_ref[...] = (
        jnp.dot(agg, w_ref[...], preferred_element_type=jnp.float32) + b_ref[...]
    )


def _tc_project(partials, W, b2d):
    return pl.pallas_call(
        _tc_body,
        out_shape=jax.ShapeDtypeStruct((N_NODES, D), jnp.float32),
        grid=(N_NODES // _BLK,),
        in_specs=[
            pl.BlockSpec((NC, _BLK, D), lambda i: (0, i, 0)),
            pl.BlockSpec((D, D), lambda i: (0, 0)),
            pl.BlockSpec((1, D), lambda i: (0, 0)),
        ],
        out_specs=pl.BlockSpec((_BLK, D), lambda i: (i, 0)),
    )(partials, W, b2d)


def kernel(x, edge_index, W, b):
    partials = _sc_aggregate(x, edge_index.astype(jnp.int32))
    return _tc_project(partials, W, b.reshape(1, D))
